# Initial kernel scaffold; baseline (speedup 1.0000x reference)
import jax
import jax.numpy as jnp
from jax.experimental import pallas as pl


def kernel(n, edge_index, e, p, W1s, W1n, b1, W2s, W2n, b2, gn_gamma, gn_beta, Wg, bg, f_gamma, f_beta, Wf1, bf1, Wf2, bf2, Wf3, bf3):
    raise NotImplementedError("write your pallas kernel here")


# speedup vs baseline: 7.2356x; 7.2356x over previous
import functools

import jax
import jax.numpy as jnp
from jax import lax
from jax.experimental import pallas as pl
from jax.experimental.pallas import tpu as pltpu, tpu_sc as plsc

N = 10000
E = 320000
HID = 16
NPAD = 10112
NC, NS = 2, 16
NW = NC * NS
EPW = E // NW
CH = 80
NCHUNK = EPW // CH
ROWS_PER_SUB = NPAD // NS


def _make_sc_pass(width):
    mesh = plsc.VectorSubcoreMesh(core_axis_name="c", subcore_axis_name="s")

    @functools.partial(
        pl.kernel,
        out_type=jax.ShapeDtypeStruct((NC * NPAD, width), jnp.float32),
        mesh=mesh,
        scratch_types=[
            pltpu.VMEM((CH,), jnp.int32),
            pltpu.VMEM((CH,), jnp.int32),
            pltpu.VMEM((CH, width), jnp.float32),
            pltpu.VMEM_SHARED((NPAD, width), jnp.float32),
            pltpu.SemaphoreType.DMA,
        ],
        compiler_params=pltpu.CompilerParams(use_tc_tiling_on_sc=False),
    )
    def sc_pass(src_hbm, dst_hbm, table_hbm, zeros_hbm, out_hbm,
                sidx, didx, rows, acc, sem):
        c = lax.axis_index("c")
        s = lax.axis_index("s")
        wid = c * NS + s
        pltpu.sync_copy(zeros_hbm, acc.at[pl.ds(s * ROWS_PER_SUB, ROWS_PER_SUB)])
        plsc.subcore_barrier()

        base = wid * EPW

        def body(i, carry):
            off = pl.multiple_of(base + i * CH, 8)
            pltpu.sync_copy(src_hbm.at[pl.ds(off, CH)], sidx)
            pltpu.sync_copy(dst_hbm.at[pl.ds(off, CH)], didx)
            pltpu.async_copy(table_hbm.at[sidx], rows, sem).wait()
            pltpu.sync_copy(rows, acc.at[didx], add=True)
            return carry

        lax.fori_loop(0, NCHUNK, body, 0)
        plsc.subcore_barrier()
        pltpu.sync_copy(
            acc.at[pl.ds(s * ROWS_PER_SUB, ROWS_PER_SUB)],
            out_hbm.at[pl.ds(c * NPAD + s * ROWS_PER_SUB, ROWS_PER_SUB)],
        )

    return sc_pass


_sc_pass32 = _make_sc_pass(32)
_sc_pass16 = _make_sc_pass(HID)


def _tc1(n_ref, w_ref, b1_ref, table_ref, s1_ref):
    t = jnp.dot(n_ref[...], w_ref[...], preferred_element_type=jnp.float32)
    y1n = t[:, :HID]
    table_ref[...] = jnp.concatenate([y1n, jnp.ones_like(y1n)], axis=1)
    s1_ref[...] = t[:, HID:] + b1_ref[...]


def _tc2(acc_ref, s1_ref, w2_ref, b2_ref, table2_ref, s2_ref, degc_ref):
    a = acc_ref[0:N, :] + acc_ref[NPAD:NPAD + N, :]
    sum1 = a[:, :HID]
    degc = jnp.clip(a[:, HID:], 1.0, None)
    h = s1_ref[...] + sum1 / degc
    h = jnp.where(h > 0, h, jnp.exp(h) - 1.0)
    t2 = jnp.dot(h, w2_ref[...], preferred_element_type=jnp.float32)
    table2_ref[...] = t2[:, :HID]
    s2_ref[...] = t2[:, HID:] + b2_ref[...]
    degc_ref[...] = degc


def _tc3(acc2_ref, s2_ref, degc_ref, gg_ref, gb_ref, wg_ref, bg_ref,
         fg_ref, fb_ref, wf1_ref, bf1_ref, wf2_ref, bf2_ref, wf3_ref, bf3_ref,
         out_ref):
    sum2 = acc2_ref[0:N, :] + acc2_ref[NPAD:NPAD + N, :]
    h = s2_ref[...] + sum2 / degc_ref[...]
    inv = 1.0 / jnp.sqrt(1.0 + 1e-5)
    hbn = h * (gg_ref[...] * inv) + gb_ref[...]
    gate = jnp.sum(hbn * wg_ref[...], axis=1, keepdims=True) + bg_ref[...]
    m = jnp.max(gate)
    ex = jnp.exp(gate - m)
    z = jnp.sum(ex)
    h1 = jnp.sum(ex * h, axis=0, keepdims=True) / z
    hmax = jnp.max(h, axis=0, keepdims=True)
    hc = jnp.concatenate([h1, hmax], axis=1)
    hc = jnp.where(hc > 0, hc, jnp.exp(hc) - 1.0)
    y = hc * (fg_ref[...] * inv) + fb_ref[...]
    y = jnp.maximum(jnp.dot(y, wf1_ref[...], preferred_element_type=jnp.float32)
                    + bf1_ref[...], 0.0)
    y = jnp.maximum(jnp.dot(y, wf2_ref[...], preferred_element_type=jnp.float32)
                    + bf2_ref[...], 0.0)
    out_ref[...] = (jnp.dot(y, wf3_ref[...], preferred_element_type=jnp.float32)
                    + bf3_ref[...])


def kernel(n, edge_index, e, p, W1s, W1n, b1, W2s, W2n, b2,
           gn_gamma, gn_beta, Wg, bg, f_gamma, f_beta,
           Wf1, bf1, Wf2, bf2, Wf3, bf3):
    src = edge_index[0]
    dst = edge_index[1]
    w1cat = jnp.concatenate([W1n, W1s], axis=1)
    w2cat = jnp.concatenate([W2n, W2s], axis=1)

    table1, s1 = pl.pallas_call(
        _tc1,
        out_shape=(jax.ShapeDtypeStruct((N, 32), jnp.float32),
                   jax.ShapeDtypeStruct((N, HID), jnp.float32)),
    )(n, w1cat, b1.reshape(1, HID))

    acc1 = _sc_pass32(src, dst, table1, jnp.zeros((ROWS_PER_SUB, 32), jnp.float32))

    table2, s2, degc = pl.pallas_call(
        _tc2,
        out_shape=(jax.ShapeDtypeStruct((N, HID), jnp.float32),
                   jax.ShapeDtypeStruct((N, HID), jnp.float32),
                   jax.ShapeDtypeStruct((N, HID), jnp.float32)),
    )(acc1, s1, w2cat, b2.reshape(1, HID))

    acc2 = _sc_pass16(src, dst, table2, jnp.zeros((ROWS_PER_SUB, HID), jnp.float32))

    out = pl.pallas_call(
        _tc3,
        out_shape=jax.ShapeDtypeStruct((1, 1), jnp.float32),
    )(acc2, s2, degc,
      gn_gamma.reshape(1, HID), gn_beta.reshape(1, HID),
      Wg.reshape(1, HID), bg.reshape(1, 1),
      f_gamma.reshape(1, 32), f_beta.reshape(1, 32),
      Wf1, bf1.reshape(1, 32), Wf2, bf2.reshape(1, 32),
      Wf3, bf3.reshape(1, 1))
    return out


# speedup vs baseline: 19.2473x; 2.6601x over previous
import functools

import jax
import jax.numpy as jnp
from jax import lax
from jax.experimental import pallas as pl
from jax.experimental.pallas import tpu as pltpu, tpu_sc as plsc

N = 10000
E = 320000
HID = 16
NPAD = 10112
NC, NS = 2, 16
NW = NC * NS
EPW = E // NW
CH = 80
NCHUNK = EPW // CH
ROWS_PER_SUB = NPAD // NS


def _make_sc_pass(width):
    mesh = plsc.VectorSubcoreMesh(core_axis_name="c", subcore_axis_name="s")

    @functools.partial(
        pl.kernel,
        out_type=jax.ShapeDtypeStruct((NC * NPAD, width), jnp.float32),
        mesh=mesh,
        scratch_types=[
            pltpu.VMEM((NCHUNK, CH), jnp.int32),
            pltpu.VMEM((NCHUNK, CH), jnp.int32),
            pltpu.VMEM((CH, width), jnp.float32),
            pltpu.VMEM_SHARED((NPAD, width), jnp.float32),
            pltpu.VMEM_SHARED((NPAD, width), jnp.float32),
            pltpu.SemaphoreType.DMA,
        ],
        compiler_params=pltpu.CompilerParams(use_tc_tiling_on_sc=False),
    )
    def sc_pass(src_hbm, dst_hbm, table_hbm, zeros_hbm, out_hbm,
                sidx, didx, rows, tbl, acc, sem):
        c = lax.axis_index("c")
        s = lax.axis_index("s")
        wid = c * NS + s
        sl = pl.ds(s * ROWS_PER_SUB, ROWS_PER_SUB)
        pltpu.sync_copy(table_hbm.at[sl], tbl.at[sl])
        pltpu.sync_copy(zeros_hbm, acc.at[sl])
        pltpu.sync_copy(src_hbm.at[wid], sidx)
        pltpu.sync_copy(dst_hbm.at[wid], didx)
        plsc.subcore_barrier()

        def body(i, carry):
            pltpu.async_copy(tbl.at[sidx.at[i]], rows, sem).wait()
            pltpu.sync_copy(rows, acc.at[didx.at[i]], add=True)
            return carry

        lax.fori_loop(0, NCHUNK, body, 0)
        plsc.subcore_barrier()
        pltpu.sync_copy(acc.at[sl], out_hbm.at[pl.ds(c * NPAD + s * ROWS_PER_SUB,
                                                     ROWS_PER_SUB)])

    return sc_pass


_sc_pass32 = _make_sc_pass(32)
_sc_pass16 = _make_sc_pass(HID)


def _tc1(n_ref, w_ref, b1_ref, table_ref, s1_ref):
    t = jnp.dot(n_ref[...], w_ref[...], preferred_element_type=jnp.float32)
    y1n = t[:, :HID]
    table_ref[...] = jnp.concatenate([y1n, jnp.ones_like(y1n)], axis=1)
    s1_ref[...] = t[:, HID:] + b1_ref[...]


def _tc2(acc_ref, s1_ref, w2_ref, b2_ref, table2_ref, s2_ref, degc_ref):
    a = acc_ref[0:NPAD, :] + acc_ref[NPAD:, :]
    sum1 = a[:, :HID]
    degc = jnp.clip(a[:, HID:], 1.0, None)
    h = s1_ref[...] + sum1 / degc
    h = jnp.where(h > 0, h, jnp.exp(h) - 1.0)
    t2 = jnp.dot(h, w2_ref[...], preferred_element_type=jnp.float32)
    table2_ref[...] = t2[:, :HID]
    s2_ref[...] = t2[:, HID:] + b2_ref[...]
    degc_ref[...] = degc


def _tc3(acc2_ref, s2_ref, degc_ref, gg_ref, gb_ref, wg_ref, bg_ref,
         fg_ref, fb_ref, wf1_ref, bf1_ref, wf2_ref, bf2_ref, wf3_ref, bf3_ref,
         out_ref):
    sum2 = acc2_ref[0:N, :] + acc2_ref[NPAD:NPAD + N, :]
    h = s2_ref[0:N, :] + sum2 / degc_ref[0:N, :]
    inv = 1.0 / jnp.sqrt(1.0 + 1e-5)
    hbn = h * (gg_ref[...] * inv) + gb_ref[...]
    gate = jnp.sum(hbn * wg_ref[...], axis=1, keepdims=True) + bg_ref[...]
    m = jnp.max(gate)
    ex = jnp.exp(gate - m)
    z = jnp.sum(ex)
    h1 = jnp.sum(ex * h, axis=0, keepdims=True) / z
    hmax = jnp.max(h, axis=0, keepdims=True)
    hc = jnp.concatenate([h1, hmax], axis=1)
    hc = jnp.where(hc > 0, hc, jnp.exp(hc) - 1.0)
    y = hc * (fg_ref[...] * inv) + fb_ref[...]
    y = jnp.maximum(jnp.dot(y, wf1_ref[...], preferred_element_type=jnp.float32)
                    + bf1_ref[...], 0.0)
    y = jnp.maximum(jnp.dot(y, wf2_ref[...], preferred_element_type=jnp.float32)
                    + bf2_ref[...], 0.0)
    out_ref[...] = (jnp.dot(y, wf3_ref[...], preferred_element_type=jnp.float32)
                    + bf3_ref[...])


def kernel(n, edge_index, e, p, W1s, W1n, b1, W2s, W2n, b2,
           gn_gamma, gn_beta, Wg, bg, f_gamma, f_beta,
           Wf1, bf1, Wf2, bf2, Wf3, bf3):
    src = edge_index[0].reshape(NW, NCHUNK, CH)
    dst = edge_index[1].reshape(NW, NCHUNK, CH)
    npad = jnp.pad(n, ((0, NPAD - N), (0, 0)))
    w1cat = jnp.concatenate([W1n, W1s], axis=1)
    w2cat = jnp.concatenate([W2n, W2s], axis=1)

    table1, s1 = pl.pallas_call(
        _tc1,
        out_shape=(jax.ShapeDtypeStruct((NPAD, 32), jnp.float32),
                   jax.ShapeDtypeStruct((NPAD, HID), jnp.float32)),
    )(npad, w1cat, b1.reshape(1, HID))

    acc1 = _sc_pass32(src, dst, table1, jnp.zeros((ROWS_PER_SUB, 32), jnp.float32))

    table2, s2, degc = pl.pallas_call(
        _tc2,
        out_shape=(jax.ShapeDtypeStruct((NPAD, HID), jnp.float32),
                   jax.ShapeDtypeStruct((NPAD, HID), jnp.float32),
                   jax.ShapeDtypeStruct((NPAD, HID), jnp.float32)),
    )(acc1, s1, w2cat, b2.reshape(1, HID))

    acc2 = _sc_pass16(src, dst, table2, jnp.zeros((ROWS_PER_SUB, HID), jnp.float32))

    out = pl.pallas_call(
        _tc3,
        out_shape=jax.ShapeDtypeStruct((1, 1), jnp.float32),
    )(acc2, s2, degc,
      gn_gamma.reshape(1, HID), gn_beta.reshape(1, HID),
      Wg.reshape(1, HID), bg.reshape(1, 1),
      f_gamma.reshape(1, 32), f_beta.reshape(1, 32),
      Wf1, bf1.reshape(1, 32), Wf2, bf2.reshape(1, 32),
      Wf3, bf3.reshape(1, 1))
    return out
